# Initial kernel scaffold; baseline (speedup 1.0000x reference)
import jax
import jax.numpy as jnp
from jax.experimental import pallas as pl


def kernel(vocab_embeddings, token_en):
    raise NotImplementedError("write your pallas kernel here")


# speedup vs baseline: 4.0855x; 4.0855x over previous
import functools

import jax
import jax.numpy as jnp
from jax import lax
from jax.experimental import pallas as pl
from jax.experimental.pallas import tpu as pltpu
from jax.experimental.pallas import tpu_sc as plsc

VOCAB = 100000
EMBED_DIM = 64
BATCH = 4096
HIST = 50

try:
    _INFO = plsc.get_sparse_core_info()
    _NC = _INFO.num_cores
    _NS = _INFO.num_subcores
except Exception:
    _NC, _NS = 2, 16
_NW = _NC * _NS
_B = BATCH * HIST
_B_PER_W = _B // _NW
_CHUNK = 128
_NCHUNK = _B_PER_W // _CHUNK


def _make_gather():
    mesh = plsc.VectorSubcoreMesh(core_axis_name="c", subcore_axis_name="s")

    @functools.partial(
        pl.kernel,
        out_type=jax.ShapeDtypeStruct((_B, EMBED_DIM), jnp.float32),
        mesh=mesh,
        scratch_types=[
            pltpu.VMEM((_B_PER_W,), jnp.int32),
            pltpu.VMEM((_CHUNK, EMBED_DIM), jnp.float32),
            pltpu.SemaphoreType.DMA,
        ],
        compiler_params=pltpu.CompilerParams(use_tc_tiling_on_sc=False),
    )
    def gather_kernel(table_hbm, idx_hbm, out_hbm, idx_v, rows_v, sem):
        wid = lax.axis_index("s") * _NC + lax.axis_index("c")
        base = wid * _B_PER_W
        pltpu.sync_copy(idx_hbm.at[pl.ds(base, _B_PER_W)], idx_v)

        def chunk_body(j):
            off = j * _CHUNK
            pltpu.async_copy(
                table_hbm.at[idx_v.at[pl.ds(off, _CHUNK)]], rows_v, sem
            ).wait()
            pltpu.sync_copy(rows_v, out_hbm.at[pl.ds(base + off, _CHUNK)])

        lax.fori_loop(0, _NCHUNK, lambda j, c: (chunk_body(j), c)[1], 0,
                      unroll=False)

    return gather_kernel


_GATHER = _make_gather()


def kernel(vocab_embeddings, token_en):
    idx = token_en.reshape(_B).astype(jnp.int32)
    flat = _GATHER(vocab_embeddings, idx)
    return (flat.reshape(BATCH, HIST, EMBED_DIM),)


# speedup vs baseline: 4.6763x; 1.1446x over previous
import functools

import jax
import jax.numpy as jnp
from jax import lax
from jax.experimental import pallas as pl
from jax.experimental.pallas import tpu as pltpu
from jax.experimental.pallas import tpu_sc as plsc

VOCAB = 100000
EMBED_DIM = 64
BATCH = 4096
HIST = 50

try:
    _INFO = plsc.get_sparse_core_info()
    _NC = _INFO.num_cores
    _NS = _INFO.num_subcores
except Exception:
    _NC, _NS = 2, 16

_NW = _NC * _NS
_B = BATCH * HIST
_B_PER_W = _B // _NW
_CHUNK = 400
_NCHUNK = _B_PER_W // _CHUNK
_K = 4
assert _NCHUNK % _K == 0 and _CHUNK % 8 == 0


def _make_gather():
    mesh = plsc.VectorSubcoreMesh(core_axis_name="c", subcore_axis_name="s")

    @functools.partial(
        pl.kernel,
        out_type=jax.ShapeDtypeStruct((_B, EMBED_DIM), jnp.float32),
        mesh=mesh,
        scratch_types=(
            [pltpu.VMEM((_B_PER_W,), jnp.int32)]
            + [pltpu.VMEM((_CHUNK, EMBED_DIM), jnp.float32) for _ in range(_K)]
            + [pltpu.SemaphoreType.DMA for _ in range(2 * _K)]
        ),
        compiler_params=pltpu.CompilerParams(use_tc_tiling_on_sc=False),
    )
    def gather_kernel(table_hbm, idx_hbm, out_hbm, idx_v, *scr):
        bufs = scr[:_K]
        gsems = scr[_K:2 * _K]
        ssems = scr[2 * _K:3 * _K]
        wid = lax.axis_index("s") * _NC + lax.axis_index("c")
        base = wid * _B_PER_W
        pltpu.sync_copy(idx_hbm.at[pl.ds(base, _B_PER_W)], idx_v)

        def fire_gather(g, b):
            pltpu.async_copy(
                table_hbm.at[idx_v.at[pl.ds(g * _CHUNK, _CHUNK)]],
                bufs[b], gsems[b])

        def wait_gather(b):
            pltpu.make_async_copy(
                table_hbm.at[pl.ds(0, _CHUNK)], bufs[b], gsems[b]).wait()

        def fire_scatter(g, b):
            pltpu.async_copy(
                bufs[b], out_hbm.at[pl.ds(base + g * _CHUNK, _CHUNK)],
                ssems[b])

        def wait_scatter(b):
            pltpu.make_async_copy(
                bufs[b], out_hbm.at[pl.ds(base, _CHUNK)], ssems[b]).wait()

        for b in range(_K - 1):
            fire_gather(b, b)
        fire_gather(_K - 1, _K - 1)
        wait_gather(0)
        fire_scatter(0, 0)

        @pl.loop(0, (_NCHUNK - _K) // _K)
        def _steady(i):
            for b in range(_K):
                g = 1 + i * _K + b
                wait_scatter(b)
                fire_gather(g + _K - 1, b)
                wait_gather((b + 1) % _K)
                fire_scatter(g, (b + 1) % _K)

        for g in range(_NCHUNK - _K + 1, _NCHUNK):
            wait_gather(g % _K)
            fire_scatter(g, g % _K)
        for b in range(_K):
            wait_scatter(b)

    return gather_kernel


_GATHER = _make_gather()


def kernel(vocab_embeddings, token_en):
    idx = token_en.reshape(_B).astype(jnp.int32)
    flat = _GATHER(vocab_embeddings, idx)
    return (flat.reshape(BATCH, HIST, EMBED_DIM),)
